# baseline (speedup 1.0000x reference)
import functools

import jax
import jax.numpy as jnp
from jax import lax
from jax.experimental import pallas as pl
from jax.experimental.pallas import tpu as pltpu
from jax.experimental.pallas import tpu_sc as plsc

T = 2048
D = 1024
F = 2048
E = 8
K = 2

BT = 256
A = T * K
A_PAD = A + E * BT
NBLK = A_PAD // BT

NC = 2
NS = 16
NW = NC * NS
TOK_W = T // NW


def _cumsum_lanes(m):
    c = m
    d = 1
    n = m.shape[1]
    while d < n:
        z = jnp.zeros((m.shape[0], d), m.dtype)
        c = c + jnp.concatenate([z, c[:, :-d]], axis=1)
        d *= 2
    return c


def _router_body(x_ref, wg_ref, slots_ref, gsp_ref, be_ref, act_ref,
                 first_ref, nxt_ref, hn_ref, brank_ref, xpk_ref):
    x = x_ref[...]
    wg = wg_ref[...]
    logits = jnp.dot(x, wg, preferred_element_type=jnp.float32)

    xb16 = x.astype(jnp.bfloat16)
    a16 = jax.lax.bitcast_convert_type(xb16[:, :D // 2], jnp.uint16)
    b16 = jax.lax.bitcast_convert_type(xb16[:, D // 2:], jnp.uint16)
    xpk_ref[...] = a16.astype(jnp.int32) | (b16.astype(jnp.int32) << 16)
    lt = logits.T

    er = jax.lax.broadcasted_iota(jnp.int32, (E, T), 0)
    neg_inf = jnp.float32(-jnp.inf)

    v0 = jnp.max(lt, axis=0, keepdims=True)
    idx0 = jnp.min(jnp.where(lt == v0, er, E), axis=0, keepdims=True)
    m0 = er == idx0
    masked = jnp.where(m0, neg_inf, lt)
    v1 = jnp.max(masked, axis=0, keepdims=True)
    idx1 = jnp.min(jnp.where(masked == v1, er, E), axis=0, keepdims=True)
    m1 = er == idx1

    g1 = 1.0 / (1.0 + jnp.exp(v0 - v1))
    g0 = 1.0 - g1

    mi0 = m0.astype(jnp.int32)
    mi1 = m1.astype(jnp.int32)
    cum0 = _cumsum_lanes(mi0)
    cum1 = _cumsum_lanes(mi1)
    tot0 = cum0[:, -1:]
    count = tot0 + cum1[:, -1:]

    pc = ((count + (BT - 1)) // BT) * BT
    b = pc
    d = 1
    while d < E:
        b = b + jnp.concatenate([jnp.zeros((d, 1), jnp.int32), b[:-d, :]],
                                axis=0)
        d *= 2
    base = b - pc

    rank0 = cum0 - 1
    rank1 = tot0 + cum1 - 1
    slot0 = jnp.sum(mi0 * (base + rank0), axis=0, keepdims=True)
    slot1 = jnp.sum(mi1 * (base + rank1), axis=0, keepdims=True)

    slots_ref[0:1, :] = slot0
    slots_ref[1:2, :] = slot1
    gsp_ref[0] = jnp.broadcast_to(g0.T, (T, 16))
    gsp_ref[1] = jnp.broadcast_to(g1.T, (T, 16))

    blk = jax.lax.broadcasted_iota(jnp.int32, (NBLK, E), 0) * BT
    seg_end = jnp.broadcast_to((base + pc).reshape(1, E), (NBLK, E))
    base_b = jnp.broadcast_to(base.reshape(1, E), (NBLK, E))
    pc_b = jnp.broadcast_to(pc.reshape(1, E), (NBLK, E))
    be_raw = jnp.sum((seg_end <= blk).astype(jnp.int32), axis=1)
    be = jnp.minimum(be_raw, E - 1)
    be_ref[...] = be
    total_padded = jnp.sum(pc)
    blk1 = jax.lax.iota(jnp.int32, NBLK) * BT
    act = (blk1 < total_padded).astype(jnp.int32)
    act_ref[...] = act

    nonempty = (pc_b > 0).astype(jnp.int32)
    first = (jnp.sum(((blk == base_b) * nonempty), axis=1) > 0)
    first_ref[...] = first.astype(jnp.int32) * act
    my_end = jnp.sum(jnp.where(
        jax.lax.broadcasted_iota(jnp.int32, (NBLK, E), 1) == be[:, None],
        seg_end, 0), axis=1)
    nxt_raw = jnp.sum((seg_end <= my_end[:, None]).astype(jnp.int32), axis=1)
    nxt_ref[...] = jnp.minimum(nxt_raw, E - 1)
    hn_ref[...] = (nxt_raw < E).astype(jnp.int32) * act
    brank_ref[...] = jnp.sum(((base_b <= blk) * nonempty), axis=1)


def _router(x, Wg):
    return pl.pallas_call(
        _router_body,
        out_shape=(
            jax.ShapeDtypeStruct((K, T), jnp.int32),
            jax.ShapeDtypeStruct((K, T, 16), jnp.float32),
            jax.ShapeDtypeStruct((NBLK,), jnp.int32),
            jax.ShapeDtypeStruct((NBLK,), jnp.int32),
            jax.ShapeDtypeStruct((NBLK,), jnp.int32),
            jax.ShapeDtypeStruct((NBLK,), jnp.int32),
            jax.ShapeDtypeStruct((NBLK,), jnp.int32),
            jax.ShapeDtypeStruct((NBLK,), jnp.int32),
            jax.ShapeDtypeStruct((T, D // 2), jnp.int32),
        ),
    )(x, Wg)


@functools.partial(
    pl.kernel,
    out_type=jax.ShapeDtypeStruct((A_PAD, D // 2), jnp.int32),
    mesh=plsc.VectorSubcoreMesh(core_axis_name="c", subcore_axis_name="s",
                                num_cores=NC, num_subcores=NS),
    scratch_types=[
        pltpu.VMEM((TOK_W,), jnp.int32),
        pltpu.VMEM((TOK_W,), jnp.int32),
        pltpu.VMEM((TOK_W, D // 2), jnp.int32),
        pltpu.SemaphoreType.DMA,
    ],
)
def _dispatch(x_hbm, slot0_hbm, slot1_hbm, xg_hbm, idx0_v, idx1_v, rows_v,
              sem):
    wid = lax.axis_index("s") * NC + lax.axis_index("c")
    base = wid * TOK_W
    pltpu.sync_copy(x_hbm.at[pl.ds(base, TOK_W)], rows_v)
    pltpu.sync_copy(slot0_hbm.at[pl.ds(base, TOK_W)], idx0_v)
    pltpu.sync_copy(slot1_hbm.at[pl.ds(base, TOK_W)], idx1_v)
    c0 = pltpu.async_copy(rows_v, xg_hbm.at[idx0_v], sem)
    c1 = pltpu.async_copy(rows_v, xg_hbm.at[idx1_v], sem)
    c0.wait()
    c1.wait()


CHUNK = 32


@functools.partial(
    pl.kernel,
    out_type=jax.ShapeDtypeStruct((T, D), jnp.float32),
    mesh=plsc.VectorSubcoreMesh(core_axis_name="c", subcore_axis_name="s",
                                num_cores=NC, num_subcores=NS),
    scratch_types=[
        pltpu.VMEM((CHUNK,), jnp.int32),
        pltpu.VMEM((CHUNK,), jnp.int32),
        pltpu.VMEM((CHUNK, 16), jnp.float32),
        pltpu.VMEM((CHUNK, 16), jnp.float32),
        pltpu.VMEM((CHUNK, D), jnp.float32),
        pltpu.VMEM((CHUNK, D), jnp.float32),
        pltpu.VMEM((CHUNK, D), jnp.float32),
        pltpu.SemaphoreType.DMA,
    ],
)
def _combine(y_hbm, slot0_hbm, slot1_hbm, g0sp_hbm, g1sp_hbm, out_hbm,
             idx0_v, idx1_v, g0_v, g1_v, y0_v, y1_v, out_v, sem):
    wid = lax.axis_index("s") * NC + lax.axis_index("c")
    for chunk in range(TOK_W // CHUNK):
        tb = wid * TOK_W + chunk * CHUNK
        pltpu.sync_copy(slot0_hbm.at[pl.ds(tb, CHUNK)], idx0_v)
        pltpu.sync_copy(slot1_hbm.at[pl.ds(tb, CHUNK)], idx1_v)
        pltpu.sync_copy(g0sp_hbm.at[pl.ds(tb, CHUNK)], g0_v)
        pltpu.sync_copy(g1sp_hbm.at[pl.ds(tb, CHUNK)], g1_v)
        c0 = pltpu.async_copy(y_hbm.at[idx0_v], y0_v, sem)
        c1 = pltpu.async_copy(y_hbm.at[idx1_v], y1_v, sem)
        c0.wait()
        c1.wait()

        def body(j, carry):
            g0j = g0_v[j]
            g1j = g1_v[j]
            for c in range(D // 16):
                sl = pl.ds(c * 16, 16)
                out_v[j, sl] = g0j * y0_v[j, sl] + g1j * y1_v[j, sl]
            return carry

        lax.fori_loop(0, CHUNK, body, 0, unroll=4)
        pltpu.sync_copy(out_v, out_hbm.at[pl.ds(tb, CHUNK)])


def _expert_body(be_ref, act_ref, first_ref, nxt_ref, hn_ref, brank_ref,
                 xg_ref, w1_hbm, w2_hbm, y_ref,
                 w1_sc, w2_sc, sem1, sem2):
    i = pl.program_id(0)
    buf = (brank_ref[i] - 1) & 1

    def issue(e, b):
        pltpu.make_async_copy(w1_hbm.at[e], w1_sc.at[b], sem1.at[b]).start()
        pltpu.make_async_copy(w2_hbm.at[e], w2_sc.at[b], sem2.at[b]).start()

    def drain(e, b):
        pltpu.make_async_copy(w1_hbm.at[e], w1_sc.at[b], sem1.at[b]).wait()
        pltpu.make_async_copy(w2_hbm.at[e], w2_sc.at[b], sem2.at[b]).wait()

    @pl.when(i == 0)
    def _():
        issue(be_ref[0], 0)

    @pl.when((first_ref[i] == 1) & (hn_ref[i] == 1))
    def _():
        issue(nxt_ref[i], 1 - buf)

    @pl.when(first_ref[i] == 1)
    def _():
        drain(be_ref[i], buf)

    @pl.when(act_ref[i] == 1)
    def _():
        w = xg_ref[...]
        lo = jax.lax.bitcast_convert_type(w << 16, jnp.float32)
        hi = jax.lax.bitcast_convert_type(w & jnp.int32(-65536), jnp.float32)
        xb = jnp.concatenate([lo, hi], axis=1)
        h = jnp.dot(xb, w1_sc[buf], preferred_element_type=jnp.float32)
        h = jax.nn.gelu(h)
        y_ref[...] = jnp.dot(h, w2_sc[buf], preferred_element_type=jnp.float32)


def _experts(xg, W1, W2, be, act, first, nxt, hn, brank):
    grid_spec = pltpu.PrefetchScalarGridSpec(
        num_scalar_prefetch=6,
        grid=(NBLK,),
        in_specs=[
            pl.BlockSpec((BT, D // 2),
                         lambda i, be, act, *_:
                         (jnp.where(act[i] == 1, i, NBLK - 1), 0)),
            pl.BlockSpec(memory_space=pl.ANY),
            pl.BlockSpec(memory_space=pl.ANY),
        ],
        out_specs=pl.BlockSpec((BT, D),
                               lambda i, be, act, *_:
                               (jnp.where(act[i] == 1, i, NBLK - 1), 0)),
        scratch_shapes=[
            pltpu.VMEM((2, D, F), jnp.float32),
            pltpu.VMEM((2, F, D), jnp.float32),
            pltpu.SemaphoreType.DMA((2,)),
            pltpu.SemaphoreType.DMA((2,)),
        ],
    )
    return pl.pallas_call(
        _expert_body,
        grid_spec=grid_spec,
        out_shape=jax.ShapeDtypeStruct((A_PAD, D), jnp.float32),
    )(be, act, first, nxt, hn, brank, xg, W1, W2)


@jax.jit
def kernel(x, Wg, W1, W2):
    slots, gsp, be, act, first, nxt, hn, brank, xpk = _router(x, Wg)
    slot0, slot1 = slots[0], slots[1]

    xg = _dispatch(xpk, slot0, slot1)
    y = _experts(xg, W1, W2, be, act, first, nxt, hn, brank)
    return _combine(y, slot0, slot1, gsp[0], gsp[1])


# speedup vs baseline: 1.0437x; 1.0437x over previous
import functools

import jax
import jax.numpy as jnp
from jax import lax
from jax.experimental import pallas as pl
from jax.experimental.pallas import tpu as pltpu
from jax.experimental.pallas import tpu_sc as plsc

T = 2048
D = 1024
F = 2048
E = 8
K = 2

BT = 256
A = T * K
A_PAD = A + E * BT
NBLK = A_PAD // BT

NC = 2
NS = 16
NW = NC * NS
TOK_W = T // NW


def _cumsum_lanes(m):
    c = m
    d = 1
    n = m.shape[1]
    while d < n:
        z = jnp.zeros((m.shape[0], d), m.dtype)
        c = c + jnp.concatenate([z, c[:, :-d]], axis=1)
        d *= 2
    return c


def _router_body(x_ref, wg_ref, slots_ref, gsp_ref, be_ref, act_ref,
                 first_ref, nxt_ref, hn_ref, brank_ref, xpk_ref):
    x = x_ref[...]
    wg = wg_ref[...]
    logits = jnp.dot(x, wg, preferred_element_type=jnp.float32)

    xb16 = x.astype(jnp.bfloat16)
    a16 = jax.lax.bitcast_convert_type(xb16[:, :D // 2], jnp.uint16)
    b16 = jax.lax.bitcast_convert_type(xb16[:, D // 2:], jnp.uint16)
    xpk_ref[...] = a16.astype(jnp.int32) | (b16.astype(jnp.int32) << 16)
    lt = logits.T

    er = jax.lax.broadcasted_iota(jnp.int32, (E, T), 0)
    neg_inf = jnp.float32(-jnp.inf)

    v0 = jnp.max(lt, axis=0, keepdims=True)
    idx0 = jnp.min(jnp.where(lt == v0, er, E), axis=0, keepdims=True)
    m0 = er == idx0
    masked = jnp.where(m0, neg_inf, lt)
    v1 = jnp.max(masked, axis=0, keepdims=True)
    idx1 = jnp.min(jnp.where(masked == v1, er, E), axis=0, keepdims=True)
    m1 = er == idx1

    g1 = 1.0 / (1.0 + jnp.exp(v0 - v1))
    g0 = 1.0 - g1

    mi0 = m0.astype(jnp.int32)
    mi1 = m1.astype(jnp.int32)
    cum0 = _cumsum_lanes(mi0)
    cum1 = _cumsum_lanes(mi1)
    tot0 = cum0[:, -1:]
    count = tot0 + cum1[:, -1:]

    pc = ((count + (BT - 1)) // BT) * BT
    b = pc
    d = 1
    while d < E:
        b = b + jnp.concatenate([jnp.zeros((d, 1), jnp.int32), b[:-d, :]],
                                axis=0)
        d *= 2
    base = b - pc

    rank0 = cum0 - 1
    rank1 = tot0 + cum1 - 1
    slot0 = jnp.sum(mi0 * (base + rank0), axis=0, keepdims=True)
    slot1 = jnp.sum(mi1 * (base + rank1), axis=0, keepdims=True)

    slots_ref[0:1, :] = slot0
    slots_ref[1:2, :] = slot1
    gsp_ref[0] = jnp.broadcast_to(g0.T, (T, 16))
    gsp_ref[1] = jnp.broadcast_to(g1.T, (T, 16))

    blk = jax.lax.broadcasted_iota(jnp.int32, (NBLK, E), 0) * BT
    seg_end = jnp.broadcast_to((base + pc).reshape(1, E), (NBLK, E))
    base_b = jnp.broadcast_to(base.reshape(1, E), (NBLK, E))
    pc_b = jnp.broadcast_to(pc.reshape(1, E), (NBLK, E))
    be_raw = jnp.sum((seg_end <= blk).astype(jnp.int32), axis=1)
    be = jnp.minimum(be_raw, E - 1)
    be_ref[...] = be
    total_padded = jnp.sum(pc)
    blk1 = jax.lax.iota(jnp.int32, NBLK) * BT
    act = (blk1 < total_padded).astype(jnp.int32)
    act_ref[...] = act

    nonempty = (pc_b > 0).astype(jnp.int32)
    first = (jnp.sum(((blk == base_b) * nonempty), axis=1) > 0)
    first_ref[...] = first.astype(jnp.int32) * act
    my_end = jnp.sum(jnp.where(
        jax.lax.broadcasted_iota(jnp.int32, (NBLK, E), 1) == be[:, None],
        seg_end, 0), axis=1)
    nxt_raw = jnp.sum((seg_end <= my_end[:, None]).astype(jnp.int32), axis=1)
    nxt_ref[...] = jnp.minimum(nxt_raw, E - 1)
    hn_ref[...] = (nxt_raw < E).astype(jnp.int32) * act
    brank_ref[...] = jnp.sum(((base_b <= blk) * nonempty), axis=1)


def _router(x, Wg):
    return pl.pallas_call(
        _router_body,
        out_shape=(
            jax.ShapeDtypeStruct((K, T), jnp.int32),
            jax.ShapeDtypeStruct((K, T, 16), jnp.float32),
            jax.ShapeDtypeStruct((NBLK,), jnp.int32),
            jax.ShapeDtypeStruct((NBLK,), jnp.int32),
            jax.ShapeDtypeStruct((NBLK,), jnp.int32),
            jax.ShapeDtypeStruct((NBLK,), jnp.int32),
            jax.ShapeDtypeStruct((NBLK,), jnp.int32),
            jax.ShapeDtypeStruct((NBLK,), jnp.int32),
            jax.ShapeDtypeStruct((T, D // 2), jnp.int32),
        ),
    )(x, Wg)


@functools.partial(
    pl.kernel,
    out_type=jax.ShapeDtypeStruct((A_PAD, D // 2), jnp.int32),
    mesh=plsc.VectorSubcoreMesh(core_axis_name="c", subcore_axis_name="s",
                                num_cores=NC, num_subcores=NS),
    scratch_types=[
        pltpu.VMEM((TOK_W,), jnp.int32),
        pltpu.VMEM((TOK_W,), jnp.int32),
        pltpu.VMEM((TOK_W, D // 2), jnp.int32),
        pltpu.SemaphoreType.DMA,
    ],
)
def _dispatch(x_hbm, slot0_hbm, slot1_hbm, xg_hbm, idx0_v, idx1_v, rows_v,
              sem):
    wid = lax.axis_index("s") * NC + lax.axis_index("c")
    base = wid * TOK_W
    pltpu.sync_copy(x_hbm.at[pl.ds(base, TOK_W)], rows_v)
    pltpu.sync_copy(slot0_hbm.at[pl.ds(base, TOK_W)], idx0_v)
    pltpu.sync_copy(slot1_hbm.at[pl.ds(base, TOK_W)], idx1_v)
    c0 = pltpu.async_copy(rows_v, xg_hbm.at[idx0_v], sem)
    c1 = pltpu.async_copy(rows_v, xg_hbm.at[idx1_v], sem)
    c0.wait()
    c1.wait()


CHUNK = 16
NCH = TOK_W // CHUNK


@functools.partial(
    pl.kernel,
    out_type=jax.ShapeDtypeStruct((T, D), jnp.float32),
    mesh=plsc.VectorSubcoreMesh(core_axis_name="c", subcore_axis_name="s",
                                num_cores=NC, num_subcores=NS),
    scratch_types=[
        pltpu.VMEM((2, CHUNK), jnp.int32),
        pltpu.VMEM((2, CHUNK), jnp.int32),
        pltpu.VMEM((CHUNK, 16), jnp.float32),
        pltpu.VMEM((CHUNK, 16), jnp.float32),
        pltpu.VMEM((2, CHUNK, D), jnp.float32),
        pltpu.VMEM((2, CHUNK, D), jnp.float32),
        pltpu.VMEM((CHUNK, D), jnp.float32),
        pltpu.SemaphoreType.DMA,
        pltpu.SemaphoreType.DMA,
    ],
)
def _combine(y_hbm, slot0_hbm, slot1_hbm, g0sp_hbm, g1sp_hbm, out_hbm,
             idx0_v, idx1_v, g0_v, g1_v, y0_v, y1_v, out_v, semA, semB):
    wid = lax.axis_index("s") * NC + lax.axis_index("c")
    sems = [semA, semB]

    def issue(chunk, b):
        tb = wid * TOK_W + chunk * CHUNK
        pltpu.sync_copy(slot0_hbm.at[pl.ds(tb, CHUNK)], idx0_v.at[b])
        pltpu.sync_copy(slot1_hbm.at[pl.ds(tb, CHUNK)], idx1_v.at[b])
        c0 = pltpu.async_copy(y_hbm.at[idx0_v.at[b]], y0_v.at[b], sems[b])
        c1 = pltpu.async_copy(y_hbm.at[idx1_v.at[b]], y1_v.at[b], sems[b])
        return (c0, c1)

    descs = issue(0, 0)
    for chunk in range(NCH):
        b = chunk % 2
        tb = wid * TOK_W + chunk * CHUNK
        nxt_descs = issue(chunk + 1, 1 - b) if chunk + 1 < NCH else None
        descs[0].wait()
        descs[1].wait()
        descs = nxt_descs
        pltpu.sync_copy(g0sp_hbm.at[pl.ds(tb, CHUNK)], g0_v)
        pltpu.sync_copy(g1sp_hbm.at[pl.ds(tb, CHUNK)], g1_v)

        def body(j, carry, b=b):
            g0j = g0_v[j]
            g1j = g1_v[j]
            for c in range(D // 16):
                sl = pl.ds(c * 16, 16)
                out_v[j, sl] = g0j * y0_v[b, j, sl] + g1j * y1_v[b, j, sl]
            return carry

        lax.fori_loop(0, CHUNK, body, 0)
        pltpu.sync_copy(out_v, out_hbm.at[pl.ds(tb, CHUNK)])


def _expert_body(be_ref, act_ref, first_ref, nxt_ref, hn_ref, brank_ref,
                 xg_ref, w1_hbm, w2_hbm, y_ref,
                 w1_sc, w2_sc, sem1, sem2):
    i = pl.program_id(0)
    buf = (brank_ref[i] - 1) & 1

    def issue(e, b):
        pltpu.make_async_copy(w1_hbm.at[e], w1_sc.at[b], sem1.at[b]).start()
        pltpu.make_async_copy(w2_hbm.at[e], w2_sc.at[b], sem2.at[b]).start()

    def drain(e, b):
        pltpu.make_async_copy(w1_hbm.at[e], w1_sc.at[b], sem1.at[b]).wait()
        pltpu.make_async_copy(w2_hbm.at[e], w2_sc.at[b], sem2.at[b]).wait()

    @pl.when(i == 0)
    def _():
        issue(be_ref[0], 0)

    @pl.when((first_ref[i] == 1) & (hn_ref[i] == 1))
    def _():
        issue(nxt_ref[i], 1 - buf)

    @pl.when(first_ref[i] == 1)
    def _():
        drain(be_ref[i], buf)

    @pl.when(act_ref[i] == 1)
    def _():
        w = xg_ref[...]
        lo = jax.lax.bitcast_convert_type(w << 16, jnp.float32)
        hi = jax.lax.bitcast_convert_type(w & jnp.int32(-65536), jnp.float32)
        xb = jnp.concatenate([lo, hi], axis=1)
        h = jnp.dot(xb, w1_sc[buf], preferred_element_type=jnp.float32)
        h = jax.nn.gelu(h)
        y_ref[...] = jnp.dot(h, w2_sc[buf], preferred_element_type=jnp.float32)


def _experts(xg, W1, W2, be, act, first, nxt, hn, brank):
    grid_spec = pltpu.PrefetchScalarGridSpec(
        num_scalar_prefetch=6,
        grid=(NBLK,),
        in_specs=[
            pl.BlockSpec((BT, D // 2),
                         lambda i, be, act, *_:
                         (jnp.where(act[i] == 1, i, NBLK - 1), 0)),
            pl.BlockSpec(memory_space=pl.ANY),
            pl.BlockSpec(memory_space=pl.ANY),
        ],
        out_specs=pl.BlockSpec((BT, D),
                               lambda i, be, act, *_:
                               (jnp.where(act[i] == 1, i, NBLK - 1), 0)),
        scratch_shapes=[
            pltpu.VMEM((2, D, F), jnp.float32),
            pltpu.VMEM((2, F, D), jnp.float32),
            pltpu.SemaphoreType.DMA((2,)),
            pltpu.SemaphoreType.DMA((2,)),
        ],
    )
    return pl.pallas_call(
        _expert_body,
        grid_spec=grid_spec,
        out_shape=jax.ShapeDtypeStruct((A_PAD, D), jnp.float32),
    )(be, act, first, nxt, hn, brank, xg, W1, W2)


@jax.jit
def kernel(x, Wg, W1, W2):
    slots, gsp, be, act, first, nxt, hn, brank, xpk = _router(x, Wg)
    slot0, slot1 = slots[0], slots[1]

    xg = _dispatch(xpk, slot0, slot1)
    y = _experts(xg, W1, W2, be, act, first, nxt, hn, brank)
    return _combine(y, slot0, slot1, gsp[0], gsp[1])


# speedup vs baseline: 1.0757x; 1.0306x over previous
import functools

import jax
import jax.numpy as jnp
from jax import lax
from jax.experimental import pallas as pl
from jax.experimental.pallas import tpu as pltpu
from jax.experimental.pallas import tpu_sc as plsc

T = 2048
D = 1024
F = 2048
E = 8
K = 2

BT = 256
A = T * K
A_PAD = A + E * BT
NBLK = A_PAD // BT

NC = 2
NS = 16
NW = NC * NS
TOK_W = T // NW


def _cumsum_lanes(m):
    c = m
    d = 1
    n = m.shape[1]
    while d < n:
        z = jnp.zeros((m.shape[0], d), m.dtype)
        c = c + jnp.concatenate([z, c[:, :-d]], axis=1)
        d *= 2
    return c


def _router_body(x_ref, wg_ref, slots_ref, gsp_ref, be_ref, act_ref,
                 first_ref, nxt_ref, hn_ref, brank_ref, xpk_ref):
    x = x_ref[...]
    wg = wg_ref[...]
    logits = jnp.dot(x, wg, preferred_element_type=jnp.float32)

    xb16 = x.astype(jnp.bfloat16)
    a16 = jax.lax.bitcast_convert_type(xb16[:, :D // 2], jnp.uint16)
    b16 = jax.lax.bitcast_convert_type(xb16[:, D // 2:], jnp.uint16)
    xpk_ref[...] = a16.astype(jnp.int32) | (b16.astype(jnp.int32) << 16)
    lt = logits.T

    er = jax.lax.broadcasted_iota(jnp.int32, (E, T), 0)
    neg_inf = jnp.float32(-jnp.inf)

    v0 = jnp.max(lt, axis=0, keepdims=True)
    idx0 = jnp.min(jnp.where(lt == v0, er, E), axis=0, keepdims=True)
    m0 = er == idx0
    masked = jnp.where(m0, neg_inf, lt)
    v1 = jnp.max(masked, axis=0, keepdims=True)
    idx1 = jnp.min(jnp.where(masked == v1, er, E), axis=0, keepdims=True)
    m1 = er == idx1

    g1 = 1.0 / (1.0 + jnp.exp(v0 - v1))
    g0 = 1.0 - g1

    mi0 = m0.astype(jnp.int32)
    mi1 = m1.astype(jnp.int32)
    cum0 = _cumsum_lanes(mi0)
    cum1 = _cumsum_lanes(mi1)
    tot0 = cum0[:, -1:]
    count = tot0 + cum1[:, -1:]

    pc = ((count + (BT - 1)) // BT) * BT
    b = pc
    d = 1
    while d < E:
        b = b + jnp.concatenate([jnp.zeros((d, 1), jnp.int32), b[:-d, :]],
                                axis=0)
        d *= 2
    base = b - pc

    rank0 = cum0 - 1
    rank1 = tot0 + cum1 - 1
    slot0 = jnp.sum(mi0 * (base + rank0), axis=0, keepdims=True)
    slot1 = jnp.sum(mi1 * (base + rank1), axis=0, keepdims=True)

    slots_ref[0:1, :] = slot0
    slots_ref[1:2, :] = slot1
    gsp_ref[0] = jnp.broadcast_to(g0.T, (T, 16))
    gsp_ref[1] = jnp.broadcast_to(g1.T, (T, 16))

    blk = jax.lax.broadcasted_iota(jnp.int32, (NBLK, E), 0) * BT
    seg_end = jnp.broadcast_to((base + pc).reshape(1, E), (NBLK, E))
    base_b = jnp.broadcast_to(base.reshape(1, E), (NBLK, E))
    pc_b = jnp.broadcast_to(pc.reshape(1, E), (NBLK, E))
    be_raw = jnp.sum((seg_end <= blk).astype(jnp.int32), axis=1)
    be = jnp.minimum(be_raw, E - 1)
    be_ref[...] = be
    total_padded = jnp.sum(pc)
    blk1 = jax.lax.iota(jnp.int32, NBLK) * BT
    act = (blk1 < total_padded).astype(jnp.int32)
    act_ref[...] = act

    nonempty = (pc_b > 0).astype(jnp.int32)
    first = (jnp.sum(((blk == base_b) * nonempty), axis=1) > 0)
    first_ref[...] = first.astype(jnp.int32) * act
    my_end = jnp.sum(jnp.where(
        jax.lax.broadcasted_iota(jnp.int32, (NBLK, E), 1) == be[:, None],
        seg_end, 0), axis=1)
    nxt_raw = jnp.sum((seg_end <= my_end[:, None]).astype(jnp.int32), axis=1)
    nxt_ref[...] = jnp.minimum(nxt_raw, E - 1)
    hn_ref[...] = (nxt_raw < E).astype(jnp.int32) * act
    brank_ref[...] = jnp.sum(((base_b <= blk) * nonempty), axis=1)


def _router(x, Wg):
    return pl.pallas_call(
        _router_body,
        out_shape=(
            jax.ShapeDtypeStruct((K, T), jnp.int32),
            jax.ShapeDtypeStruct((K, T, 16), jnp.float32),
            jax.ShapeDtypeStruct((NBLK,), jnp.int32),
            jax.ShapeDtypeStruct((NBLK,), jnp.int32),
            jax.ShapeDtypeStruct((NBLK,), jnp.int32),
            jax.ShapeDtypeStruct((NBLK,), jnp.int32),
            jax.ShapeDtypeStruct((NBLK,), jnp.int32),
            jax.ShapeDtypeStruct((NBLK,), jnp.int32),
            jax.ShapeDtypeStruct((T, D // 2), jnp.int32),
        ),
    )(x, Wg)


@functools.partial(
    pl.kernel,
    out_type=jax.ShapeDtypeStruct((A_PAD, D // 2), jnp.int32),
    mesh=plsc.VectorSubcoreMesh(core_axis_name="c", subcore_axis_name="s",
                                num_cores=NC, num_subcores=NS),
    scratch_types=[
        pltpu.VMEM((TOK_W,), jnp.int32),
        pltpu.VMEM((TOK_W,), jnp.int32),
        pltpu.VMEM((TOK_W, D // 2), jnp.int32),
        pltpu.SemaphoreType.DMA,
    ],
)
def _dispatch(x_hbm, slot0_hbm, slot1_hbm, xg_hbm, idx0_v, idx1_v, rows_v,
              sem):
    wid = lax.axis_index("s") * NC + lax.axis_index("c")
    base = wid * TOK_W
    pltpu.sync_copy(x_hbm.at[pl.ds(base, TOK_W)], rows_v)
    pltpu.sync_copy(slot0_hbm.at[pl.ds(base, TOK_W)], idx0_v)
    pltpu.sync_copy(slot1_hbm.at[pl.ds(base, TOK_W)], idx1_v)
    c0 = pltpu.async_copy(rows_v, xg_hbm.at[idx0_v], sem)
    c1 = pltpu.async_copy(rows_v, xg_hbm.at[idx1_v], sem)
    c0.wait()
    c1.wait()


CHUNK = 32


@functools.partial(
    pl.kernel,
    out_type=jax.ShapeDtypeStruct((T, D), jnp.float32),
    mesh=plsc.VectorSubcoreMesh(core_axis_name="c", subcore_axis_name="s",
                                num_cores=NC, num_subcores=NS),
    scratch_types=[
        pltpu.VMEM((CHUNK,), jnp.int32),
        pltpu.VMEM((CHUNK,), jnp.int32),
        pltpu.VMEM((CHUNK, 16), jnp.float32),
        pltpu.VMEM((CHUNK, 16), jnp.float32),
        pltpu.VMEM((CHUNK, D), jnp.float32),
        pltpu.VMEM((CHUNK, D), jnp.float32),
        pltpu.VMEM((CHUNK, D), jnp.float32),
        pltpu.SemaphoreType.DMA,
    ],
)
def _combine(y_hbm, slot0_hbm, slot1_hbm, g0sp_hbm, g1sp_hbm, out_hbm,
             idx0_v, idx1_v, g0_v, g1_v, y0_v, y1_v, out_v, sem):
    wid = lax.axis_index("s") * NC + lax.axis_index("c")
    for chunk in range(TOK_W // CHUNK):
        tb = wid * TOK_W + chunk * CHUNK
        pltpu.sync_copy(slot0_hbm.at[pl.ds(tb, CHUNK)], idx0_v)
        pltpu.sync_copy(slot1_hbm.at[pl.ds(tb, CHUNK)], idx1_v)
        pltpu.sync_copy(g0sp_hbm.at[pl.ds(tb, CHUNK)], g0_v)
        pltpu.sync_copy(g1sp_hbm.at[pl.ds(tb, CHUNK)], g1_v)
        c0 = pltpu.async_copy(y_hbm.at[idx0_v], y0_v, sem)
        c1 = pltpu.async_copy(y_hbm.at[idx1_v], y1_v, sem)
        c0.wait()
        c1.wait()

        def body(j, carry):
            g0j = g0_v[j]
            g1j = g1_v[j]
            for c in range(D // 16):
                sl = pl.ds(c * 16, 16)
                out_v[j, sl] = g0j * y0_v[j, sl] + g1j * y1_v[j, sl]
            return carry

        lax.fori_loop(0, CHUNK, body, 0)
        pltpu.sync_copy(out_v, out_hbm.at[pl.ds(tb, CHUNK)])


def _expert_body(be_ref, act_ref, first_ref, nxt_ref, hn_ref, brank_ref,
                 xg_ref, w1_hbm, w2_hbm, y_ref,
                 w1_sc, w2_sc, sem1, sem2):
    i = pl.program_id(0)
    buf = (brank_ref[i] - 1) & 1

    def issue(e, b):
        pltpu.make_async_copy(w1_hbm.at[e], w1_sc.at[b], sem1.at[b]).start()
        pltpu.make_async_copy(w2_hbm.at[e], w2_sc.at[b], sem2.at[b]).start()

    def drain(e, b):
        pltpu.make_async_copy(w1_hbm.at[e], w1_sc.at[b], sem1.at[b]).wait()
        pltpu.make_async_copy(w2_hbm.at[e], w2_sc.at[b], sem2.at[b]).wait()

    @pl.when(i == 0)
    def _():
        issue(be_ref[0], 0)

    @pl.when((first_ref[i] == 1) & (hn_ref[i] == 1))
    def _():
        issue(nxt_ref[i], 1 - buf)

    @pl.when(first_ref[i] == 1)
    def _():
        drain(be_ref[i], buf)

    @pl.when(act_ref[i] == 1)
    def _():
        w = xg_ref[...]
        lo = jax.lax.bitcast_convert_type(w << 16, jnp.float32)
        hi = jax.lax.bitcast_convert_type(w & jnp.int32(-65536), jnp.float32)
        xb = jnp.concatenate([lo, hi], axis=1)
        h = jnp.dot(xb, w1_sc[buf], preferred_element_type=jnp.float32)
        h = jax.nn.gelu(h)
        y_ref[...] = jnp.dot(h, w2_sc[buf], preferred_element_type=jnp.float32)


def _experts(xg, W1, W2, be, act, first, nxt, hn, brank):
    grid_spec = pltpu.PrefetchScalarGridSpec(
        num_scalar_prefetch=6,
        grid=(NBLK,),
        in_specs=[
            pl.BlockSpec((BT, D // 2),
                         lambda i, be, act, *_:
                         (jnp.where(act[i] == 1, i, NBLK - 1), 0)),
            pl.BlockSpec(memory_space=pl.ANY),
            pl.BlockSpec(memory_space=pl.ANY),
        ],
        out_specs=pl.BlockSpec((BT, D),
                               lambda i, be, act, *_:
                               (jnp.where(act[i] == 1, i, NBLK - 1), 0)),
        scratch_shapes=[
            pltpu.VMEM((2, D, F), jnp.float32),
            pltpu.VMEM((2, F, D), jnp.float32),
            pltpu.SemaphoreType.DMA((2,)),
            pltpu.SemaphoreType.DMA((2,)),
        ],
    )
    return pl.pallas_call(
        _expert_body,
        grid_spec=grid_spec,
        out_shape=jax.ShapeDtypeStruct((A_PAD, D), jnp.float32),
    )(be, act, first, nxt, hn, brank, xg, W1, W2)


@jax.jit
def kernel(x, Wg, W1, W2):
    slots, gsp, be, act, first, nxt, hn, brank, xpk = _router(x, Wg)
    slot0, slot1 = slots[0], slots[1]

    xg = _dispatch(xpk, slot0, slot1)
    y = _experts(xg, W1, W2, be, act, first, nxt, hn, brank)
    return _combine(y, slot0, slot1, gsp[0], gsp[1])
